# baseline (speedup 1.0000x reference)
import jax
import jax.numpy as jnp
from jax import lax
from jax.experimental import pallas as pl
from jax.experimental.pallas import tpu as pltpu
from jax.experimental.pallas import tpu_sc as plsc

N_FACTORS = 64
BATCH = 16384
NUM_WORKERS = 32
B_PER_W = BATCH // NUM_WORKERS
IDX_CHUNK = 128
N_CHUNKS = B_PER_W // IDX_CHUNK
LANES = 16
N_GROUPS = B_PER_W // LANES


def _body(sess_hbm, aid_hbm, asz_hbm, stbl_hbm, atbl_hbm, out_hbm,
          sidx, aidx, asz_v, srows, arows, out_v,
          sem_in, sem_s, sem_a):
    wid = lax.axis_index("c") * 16 + lax.axis_index("s")

    c_idx = pltpu.async_copy(sess_hbm.at[wid], sidx, sem_in)
    c_aidx = pltpu.async_copy(aid_hbm.at[wid], aidx, sem_in)
    c_asz = pltpu.async_copy(asz_hbm.at[wid], asz_v, sem_in)
    c_idx.wait()
    c_aidx.wait()
    c_asz.wait()

    copies = []
    for j in range(N_CHUNKS):
        copies.append(pltpu.async_copy(
            stbl_hbm.at[sidx.at[j]], srows.at[pl.ds(j * IDX_CHUNK, IDX_CHUNK)],
            sem_s))
        copies.append(pltpu.async_copy(
            atbl_hbm.at[aidx.at[j]], arows.at[pl.ds(j * IDX_CHUNK, IDX_CHUNK)],
            sem_a))
    for c in copies:
        c.wait()

    lane = jnp.arange(LANES, dtype=jnp.int32)

    def group_body(g, _):
        row = g * LANES + lane

        def factor_body(f, acc):
            col = jnp.full((LANES,), f, dtype=jnp.int32)
            sv = plsc.load_gather(srows, [row, col])
            av = plsc.load_gather(arows, [row, col])
            return acc + sv * av

        acc = lax.fori_loop(0, N_FACTORS, factor_body,
                            jnp.zeros((LANES,), jnp.float32))
        scale = asz_v[pl.ds(g * LANES, LANES)]
        out_v[pl.ds(g * LANES, LANES)] = acc * scale
        return 0

    lax.fori_loop(0, N_GROUPS, group_body, 0)

    pltpu.sync_copy(out_v, out_hbm.at[wid])


def kernel(session, aid, aid_size, session_table, aid_table):
    mesh = plsc.VectorSubcoreMesh(core_axis_name="c", subcore_axis_name="s")
    k = pl.kernel(
        _body,
        out_type=jax.ShapeDtypeStruct((NUM_WORKERS, B_PER_W), jnp.float32),
        mesh=mesh,
        compiler_params=pltpu.CompilerParams(needs_layout_passes=False, use_tc_tiling_on_sc=False),
        scratch_types=[
            pltpu.VMEM((N_CHUNKS, IDX_CHUNK), jnp.int32),
            pltpu.VMEM((N_CHUNKS, IDX_CHUNK), jnp.int32),
            pltpu.VMEM((B_PER_W,), jnp.float32),
            pltpu.VMEM((B_PER_W, N_FACTORS), jnp.float32),
            pltpu.VMEM((B_PER_W, N_FACTORS), jnp.float32),
            pltpu.VMEM((B_PER_W,), jnp.float32),
            pltpu.SemaphoreType.DMA,
            pltpu.SemaphoreType.DMA,
            pltpu.SemaphoreType.DMA,
        ],
    )
    sess = session.astype(jnp.int32).reshape(NUM_WORKERS, N_CHUNKS, IDX_CHUNK)
    aidr = aid.astype(jnp.int32).reshape(NUM_WORKERS, N_CHUNKS, IDX_CHUNK)
    aszr = aid_size.reshape(NUM_WORKERS, B_PER_W)
    out = k(sess, aidr, aszr, session_table, aid_table)
    return out.reshape(BATCH)
